# initial kernel scaffold (unmeasured)
import jax
import jax.numpy as jnp
from jax import lax
from jax.experimental import pallas as pl
from jax.experimental.pallas import tpu as pltpu


def kernel(
    x,
):
    def body(*refs):
        pass

    out_shape = jax.ShapeDtypeStruct(..., jnp.float32)
    return pl.pallas_call(body, out_shape=out_shape)(...)


# baseline (device time: 149923 ns/iter reference)
import jax
import jax.numpy as jnp
from jax import lax
from jax.experimental import pallas as pl
from jax.experimental.pallas import tpu as pltpu

N_DEV = 4


def kernel(x):
    m_per, n = x.shape

    def body(x_ref, out_ref, send_sems, recv_sems):
        my_pos = lax.axis_index("i")
        left = (my_pos - 1) % N_DEV
        right = (my_pos + 1) % N_DEV

        barrier_sem = pltpu.get_barrier_semaphore()
        for nbr in [left, right]:
            pl.semaphore_signal(
                barrier_sem, inc=1,
                device_id=(nbr,), device_id_type=pl.DeviceIdType.MESH,
            )
        pl.semaphore_wait(barrier_sem, 2)

        out_ref[pl.ds(my_pos * m_per, m_per), :] = x_ref[:, :]

        for h in range(N_DEV - 1):
            origin = (my_pos - h) % N_DEV
            sl = pl.ds(origin * m_per, m_per)
            rdma = pltpu.make_async_remote_copy(
                src_ref=out_ref.at[sl],
                dst_ref=out_ref.at[sl],
                send_sem=send_sems.at[h],
                recv_sem=recv_sems.at[h],
                device_id=(right,),
                device_id_type=pl.DeviceIdType.MESH,
            )
            rdma.start()
            rdma.wait()

    return pl.pallas_call(
        body,
        out_shape=jax.ShapeDtypeStruct((N_DEV * m_per, n), x.dtype),
        in_specs=[pl.BlockSpec(memory_space=pltpu.VMEM)],
        out_specs=pl.BlockSpec(memory_space=pltpu.VMEM),
        scratch_shapes=[
            pltpu.SemaphoreType.DMA((N_DEV - 1,)),
            pltpu.SemaphoreType.DMA((N_DEV - 1,)),
        ],
        compiler_params=pltpu.CompilerParams(collective_id=0),
    )(x)


# device time: 83744 ns/iter; 1.7903x vs baseline; 1.7903x over previous
import jax
import jax.numpy as jnp
from jax import lax
from jax.experimental import pallas as pl
from jax.experimental.pallas import tpu as pltpu

N_DEV = 4


def kernel(x):
    m_per, n = x.shape
    half = m_per // 2

    def body(x_ref, out_ref, fs_sems, fr_sems, bs_sems, br_sems):
        my_pos = lax.axis_index("i")
        left = (my_pos - 1) % N_DEV
        right = (my_pos + 1) % N_DEV

        barrier_sem = pltpu.get_barrier_semaphore()
        for nbr in [left, right]:
            pl.semaphore_signal(
                barrier_sem, inc=1,
                device_id=(nbr,), device_id_type=pl.DeviceIdType.MESH,
            )
        pl.semaphore_wait(barrier_sem, 2)

        out_ref[pl.ds(my_pos * m_per, m_per), :] = x_ref[:, :]

        for h in range(N_DEV - 1):
            o_f = (my_pos - h) % N_DEV
            o_b = (my_pos + h) % N_DEV
            sl_f = pl.ds(o_f * m_per, half)
            sl_b = pl.ds(o_b * m_per + half, half)
            fwd = pltpu.make_async_remote_copy(
                src_ref=out_ref.at[sl_f],
                dst_ref=out_ref.at[sl_f],
                send_sem=fs_sems.at[h],
                recv_sem=fr_sems.at[h],
                device_id=(right,),
                device_id_type=pl.DeviceIdType.MESH,
            )
            bwd = pltpu.make_async_remote_copy(
                src_ref=out_ref.at[sl_b],
                dst_ref=out_ref.at[sl_b],
                send_sem=bs_sems.at[h],
                recv_sem=br_sems.at[h],
                device_id=(left,),
                device_id_type=pl.DeviceIdType.MESH,
            )
            fwd.start()
            bwd.start()
            fwd.wait_recv()
            bwd.wait_recv()
            fwd.wait_send()
            bwd.wait_send()

    return pl.pallas_call(
        body,
        out_shape=jax.ShapeDtypeStruct((N_DEV * m_per, n), x.dtype),
        in_specs=[pl.BlockSpec(memory_space=pltpu.VMEM)],
        out_specs=pl.BlockSpec(memory_space=pltpu.VMEM),
        scratch_shapes=[
            pltpu.SemaphoreType.DMA((N_DEV - 1,)),
            pltpu.SemaphoreType.DMA((N_DEV - 1,)),
            pltpu.SemaphoreType.DMA((N_DEV - 1,)),
            pltpu.SemaphoreType.DMA((N_DEV - 1,)),
        ],
        compiler_params=pltpu.CompilerParams(collective_id=0),
    )(x)


# device time: 83456 ns/iter; 1.7964x vs baseline; 1.0035x over previous
import jax
import jax.numpy as jnp
from jax import lax
from jax.experimental import pallas as pl
from jax.experimental.pallas import tpu as pltpu

N_DEV = 4


def kernel(x):
    m_per, n = x.shape
    half = m_per // 2

    def body(x_ref, out_ref, fs_sems, fr_sems, bs_sems, br_sems):
        my_pos = lax.axis_index("i")
        left = (my_pos - 1) % N_DEV
        right = (my_pos + 1) % N_DEV

        barrier_sem = pltpu.get_barrier_semaphore()
        for nbr in [left, right]:
            pl.semaphore_signal(
                barrier_sem, inc=1,
                device_id=(nbr,), device_id_type=pl.DeviceIdType.MESH,
            )
        pl.semaphore_wait(barrier_sem, 2)

        def make(src, dst_sl, h, fwd):
            return pltpu.make_async_remote_copy(
                src_ref=src,
                dst_ref=out_ref.at[dst_sl],
                send_sem=(fs_sems if fwd else bs_sems).at[h],
                recv_sem=(fr_sems if fwd else br_sems).at[h],
                device_id=(right if fwd else left,),
                device_id_type=pl.DeviceIdType.MESH,
            )

        sl_f0 = pl.ds(my_pos * m_per, half)
        sl_b0 = pl.ds(my_pos * m_per + half, half)
        f = make(x_ref.at[pl.ds(0, half)], sl_f0, 0, True)
        b = make(x_ref.at[pl.ds(half, half)], sl_b0, 0, False)
        f.start()
        b.start()

        out_ref[pl.ds(my_pos * m_per, m_per), :] = x_ref[:, :]

        rdmas = [f, b]
        for h in range(1, N_DEV - 1):
            o_f = (my_pos - h) % N_DEV
            o_b = (my_pos + h) % N_DEV
            sl_f = pl.ds(o_f * m_per, half)
            sl_b = pl.ds(o_b * m_per + half, half)
            f_next = make(out_ref.at[sl_f], sl_f, h, True)
            b_next = make(out_ref.at[sl_b], sl_b, h, False)
            f.wait_recv()
            f_next.start()
            b.wait_recv()
            b_next.start()
            f, b = f_next, b_next
            rdmas += [f, b]

        f.wait_recv()
        b.wait_recv()
        for r in rdmas:
            r.wait_send()

    return pl.pallas_call(
        body,
        out_shape=jax.ShapeDtypeStruct((N_DEV * m_per, n), x.dtype),
        in_specs=[pl.BlockSpec(memory_space=pltpu.VMEM)],
        out_specs=pl.BlockSpec(memory_space=pltpu.VMEM),
        scratch_shapes=[
            pltpu.SemaphoreType.DMA((N_DEV - 1,)),
            pltpu.SemaphoreType.DMA((N_DEV - 1,)),
            pltpu.SemaphoreType.DMA((N_DEV - 1,)),
            pltpu.SemaphoreType.DMA((N_DEV - 1,)),
        ],
        compiler_params=pltpu.CompilerParams(collective_id=0),
    )(x)


# device time: 80571 ns/iter; 1.8608x vs baseline; 1.0358x over previous
import jax
import jax.numpy as jnp
from jax import lax
from jax.experimental import pallas as pl
from jax.experimental.pallas import tpu as pltpu

N_DEV = 4
S = 2


def kernel(x):
    m_per, n = x.shape
    half = m_per // 2
    seg = half // S

    def body(x_ref, out_ref, fs_sems, fr_sems, bs_sems, br_sems):
        my_pos = lax.axis_index("i")
        left = (my_pos - 1) % N_DEV
        right = (my_pos + 1) % N_DEV

        barrier_sem = pltpu.get_barrier_semaphore()
        for nbr in [left, right]:
            pl.semaphore_signal(
                barrier_sem, inc=1,
                device_id=(nbr,), device_id_type=pl.DeviceIdType.MESH,
            )
        pl.semaphore_wait(barrier_sem, 2)

        def make(src, dst_sl, h, s, fwd):
            return pltpu.make_async_remote_copy(
                src_ref=src,
                dst_ref=out_ref.at[dst_sl],
                send_sem=(fs_sems if fwd else bs_sems).at[h, s],
                recv_sem=(fr_sems if fwd else br_sems).at[h, s],
                device_id=(right if fwd else left,),
                device_id_type=pl.DeviceIdType.MESH,
            )

        f_prev, b_prev = [], []
        for s in range(S):
            r = make(x_ref.at[pl.ds(s * seg, seg)],
                     pl.ds(my_pos * m_per + s * seg, seg), 0, s, True)
            r.start()
            f_prev.append(r)
            r = make(x_ref.at[pl.ds(half + s * seg, seg)],
                     pl.ds(my_pos * m_per + half + s * seg, seg), 0, s, False)
            r.start()
            b_prev.append(r)

        out_ref[pl.ds(my_pos * m_per, m_per), :] = x_ref[:, :]

        rdmas = list(f_prev) + list(b_prev)
        for h in range(1, N_DEV - 1):
            o_f = (my_pos - h) % N_DEV
            o_b = (my_pos + h) % N_DEV
            f_cur, b_cur = [], []
            for s in range(S):
                sl_f = pl.ds(o_f * m_per + s * seg, seg)
                f_prev[s].wait_recv()
                r = make(out_ref.at[sl_f], sl_f, h, s, True)
                r.start()
                f_cur.append(r)
                sl_b = pl.ds(o_b * m_per + half + s * seg, seg)
                b_prev[s].wait_recv()
                r = make(out_ref.at[sl_b], sl_b, h, s, False)
                r.start()
                b_cur.append(r)
            f_prev, b_prev = f_cur, b_cur
            rdmas += f_cur + b_cur

        for s in range(S):
            f_prev[s].wait_recv()
            b_prev[s].wait_recv()
        for r in rdmas:
            r.wait_send()

    return pl.pallas_call(
        body,
        out_shape=jax.ShapeDtypeStruct((N_DEV * m_per, n), x.dtype),
        in_specs=[pl.BlockSpec(memory_space=pltpu.VMEM)],
        out_specs=pl.BlockSpec(memory_space=pltpu.VMEM),
        scratch_shapes=[
            pltpu.SemaphoreType.DMA((N_DEV - 1, S)),
            pltpu.SemaphoreType.DMA((N_DEV - 1, S)),
            pltpu.SemaphoreType.DMA((N_DEV - 1, S)),
            pltpu.SemaphoreType.DMA((N_DEV - 1, S)),
        ],
        compiler_params=pltpu.CompilerParams(collective_id=0),
    )(x)
